# baseline (speedup 1.0000x reference)
import jax
import jax.numpy as jnp
from jax import lax
from jax.experimental import pallas as pl
from jax.experimental.pallas import tpu as pltpu

K = 8
B = 128
C = 768
HW = 256


def _dense_body(w_ref, act_ref, out_ref, mask_ref):
    b = pl.program_id(0)

    @pl.when(b == 0)
    def _compute_mask():
        w = w_ref[...]
        x = w
        iota = lax.broadcasted_iota(jnp.int32, (B, C), 0)
        for _ in range(K - 1):
            m = jnp.max(x, axis=0, keepdims=True)
            eq = x == m
            first = jnp.min(jnp.where(eq, iota, B), axis=0, keepdims=True)
            x = jnp.where(iota == first, -jnp.inf, x)
        thresh = jnp.max(x, axis=0, keepdims=True)
        mask_ref[...] = jnp.where(w >= thresh, 1.0, 0.0)

    out_ref[0] = act_ref[0] * mask_ref[b][:, None]


def kernel(activations, winners):
    act = activations.reshape(B, C, HW)
    out = pl.pallas_call(
        _dense_body,
        grid=(B,),
        in_specs=[
            pl.BlockSpec((B, C), lambda b: (0, 0)),
            pl.BlockSpec((1, C, HW), lambda b: (b, 0, 0)),
        ],
        out_specs=pl.BlockSpec((1, C, HW), lambda b: (b, 0, 0)),
        out_shape=jax.ShapeDtypeStruct((B, C, HW), jnp.float32),
        scratch_shapes=[pltpu.VMEM((B, C), jnp.float32)],
    )(winners, act)
    return out.reshape(B, C, 16, 16)
